# Initial kernel scaffold; baseline (speedup 1.0000x reference)
import functools

import jax
import jax.numpy as jnp
from jax import lax
from jax.experimental import pallas as pl
from jax.experimental.pallas import tpu as pltpu
from jax.experimental.pallas import tpu_sc as plsc

_LANES = 16
_CHUNK = 128
_NUM_WORKERS = 32


def _linear_body(x_ref, wt_ref, b_ref, o_ref):
    o_ref[...] = (
        jnp.dot(x_ref[...], wt_ref[...], preferred_element_type=jnp.float32)
        + b_ref[...]
    )


def _linear(x, wt, b2d, block_rows):
    n, cin = x.shape
    cout = wt.shape[1]
    return pl.pallas_call(
        _linear_body,
        grid=(n // block_rows,),
        in_specs=[
            pl.BlockSpec((block_rows, cin), lambda i: (i, 0)),
            pl.BlockSpec((cin, cout), lambda i: (0, 0)),
            pl.BlockSpec((1, cout), lambda i: (0, 0)),
        ],
        out_specs=pl.BlockSpec((block_rows, cout), lambda i: (i, 0)),
        out_shape=jax.ShapeDtypeStruct((n, cout), jnp.float32),
    )(x, wt, b2d)


def _combine_body(tr_ref, agg_ref, cnt_ref, tbl_ref, o_ref):
    c = cnt_ref[0] + cnt_ref[1]
    emb = jnp.dot(c, tbl_ref[...], preferred_element_type=jnp.float32)
    o_ref[...] = jnp.maximum(agg_ref[0] + agg_ref[1] + tr_ref[...] + emb, 0.0)


def _combine(tr, agg, cnt3, tbl, block_rows):
    n, cout = tr.shape
    n_types = tbl.shape[0]
    return pl.pallas_call(
        _combine_body,
        grid=(n // block_rows,),
        in_specs=[
            pl.BlockSpec((block_rows, cout), lambda i: (i, 0)),
            pl.BlockSpec((2, block_rows, cout), lambda i: (0, i, 0)),
            pl.BlockSpec((2, block_rows, n_types), lambda i: (0, i, 0)),
            pl.BlockSpec((n_types, cout), lambda i: (0, 0)),
        ],
        out_specs=pl.BlockSpec((block_rows, cout), lambda i: (i, 0)),
        out_shape=jax.ShapeDtypeStruct((n, cout), jnp.float32),
    )(tr, agg, cnt3, tbl)


@functools.lru_cache(maxsize=None)
def _make_sc_scatter(npad, nchunks, cout, n_types):
    rows_per_tile = npad // 16
    cnt_len = npad * n_types
    cnt_per_tile = cnt_len // 16
    mesh = plsc.VectorSubcoreMesh(core_axis_name="c", subcore_axis_name="s")

    @functools.partial(
        pl.kernel,
        out_type=[
            jax.ShapeDtypeStruct((2, npad, cout), jnp.float32),
            jax.ShapeDtypeStruct((2, cnt_len), jnp.float32),
        ],
        mesh=mesh,
        scratch_types=[
            pltpu.VMEM((nchunks, _CHUNK), jnp.int32),
            pltpu.VMEM((nchunks, _CHUNK), jnp.int32),
            pltpu.VMEM((nchunks, _CHUNK), jnp.int32),
            pltpu.VMEM((_CHUNK,), jnp.int32),
            pltpu.VMEM((_CHUNK,), jnp.float32),
            pltpu.VMEM((_CHUNK, 128), jnp.float32),
            pltpu.VMEM((2048,), jnp.float32),
            pltpu.VMEM_SHARED((npad, 128), jnp.float32),
            pltpu.VMEM_SHARED((npad * 16,), jnp.float32),
            pltpu.SemaphoreType.DMA,
        ],
    )
    def sc_scatter(
        tr_hbm, src_hbm, tgt_hbm, et_hbm, agg_out, cnt_out,
        src_v, tgt_v, et_v, cidx_v, ones_v, rows_v, zf_v,
        agg_sh, cnt_sh, gsem,
    ):
        cid = lax.axis_index("c")
        sid = lax.axis_index("s")
        wid = cid * 16 + sid

        zero16 = jnp.zeros((_LANES,), jnp.float32)
        one16 = jnp.ones((_LANES,), jnp.float32)

        def _zf(k, carry):
            zf_v[pl.ds(k * _LANES, _LANES)] = zero16
            return carry

        lax.fori_loop(0, 2048 // _LANES, _zf, 0)

        def _zr(r, carry):
            for i in range(cout // _LANES):
                rows_v[r, pl.ds(i * _LANES, _LANES)] = zero16
            return carry

        lax.fori_loop(0, _CHUNK, _zr, 0)

        for i in range(_CHUNK // _LANES):
            ones_v[pl.ds(i * _LANES, _LANES)] = one16

        rbase = sid * rows_per_tile
        for k in range(rows_per_tile // _CHUNK):
            pltpu.sync_copy(rows_v, agg_sh.at[pl.ds(rbase + k * _CHUNK, _CHUNK)])
        cbase = sid * cnt_per_tile
        for k in range(cnt_per_tile // 2048):
            pltpu.sync_copy(zf_v, cnt_sh.at[pl.ds(cbase + k * 2048, 2048)])

        pltpu.sync_copy(src_hbm.at[pl.ds(wid * nchunks, nchunks)], src_v)
        pltpu.sync_copy(tgt_hbm.at[pl.ds(wid * nchunks, nchunks)], tgt_v)
        pltpu.sync_copy(et_hbm.at[pl.ds(wid * nchunks, nchunks)], et_v)

        plsc.subcore_barrier()

        def _chunk(j, carry):
            pltpu.async_copy(tr_hbm.at[src_v.at[j]], rows_v, gsem).wait()
            for i in range(_CHUNK // _LANES):
                sl = pl.ds(i * _LANES, _LANES)
                cidx_v[sl] = tgt_v[j, sl] * n_types + et_v[j, sl]
            pltpu.sync_copy(rows_v, agg_sh.at[tgt_v.at[j]], add=True)
            pltpu.sync_copy(ones_v, cnt_sh.at[cidx_v], add=True)
            return carry

        lax.fori_loop(0, nchunks, _chunk, 0)

        plsc.subcore_barrier()

        pltpu.sync_copy(
            agg_sh.at[pl.ds(rbase, rows_per_tile)],
            agg_out.at[cid, pl.ds(rbase, rows_per_tile)],
        )
        pltpu.sync_copy(
            cnt_sh.at[pl.ds(cbase, cnt_per_tile)],
            cnt_out.at[cid, pl.ds(cbase, cnt_per_tile)],
        )

    return sc_scatter


def kernel(node_features, edge_index, edge_type, W, b, edge_emb_table):
    n, cin = node_features.shape
    cout, _ = W.shape
    n_types = edge_emb_table.shape[0]
    e = edge_index.shape[1]

    src = edge_index[0].astype(jnp.int32)
    tgt = edge_index[1].astype(jnp.int32)
    et = edge_type.astype(jnp.int32)

    npad = -(-n // 2048) * 2048
    ept = -(-e // (_NUM_WORKERS * _CHUNK)) * _CHUNK
    epad = ept * _NUM_WORKERS
    pad = epad - e
    if pad:
        src = jnp.concatenate([src, jnp.zeros((pad,), jnp.int32)])
        tgt = jnp.concatenate([tgt, jnp.full((pad,), n, jnp.int32)])
        et = jnp.concatenate([et, jnp.zeros((pad,), jnp.int32)])
    src2 = src.reshape(epad // _CHUNK, _CHUNK)
    tgt2 = tgt.reshape(epad // _CHUNK, _CHUNK)
    et2 = et.reshape(epad // _CHUNK, _CHUNK)

    transformed = _linear(node_features, W.T, b.reshape(1, cout), block_rows=2000)

    sc_fn = _make_sc_scatter(npad, ept // _CHUNK, cout, n_types)
    agg, cnt = sc_fn(transformed, src2, tgt2, et2)

    cnt3 = cnt.reshape(2, npad, n_types)
    return _combine(transformed, agg, cnt3, edge_emb_table, block_rows=2000)


# speedup vs baseline: 4.6110x; 4.6110x over previous
import functools

import jax
import jax.numpy as jnp
from jax import lax
from jax.experimental import pallas as pl
from jax.experimental.pallas import tpu as pltpu
from jax.experimental.pallas import tpu_sc as plsc

_LANES = 16
_CHUNK = 128
_NUM_WORKERS = 32


def _linear_body(x_ref, wt_ref, b_ref, o_ref):
    o_ref[...] = (
        jnp.dot(x_ref[...], wt_ref[...], preferred_element_type=jnp.float32)
        + b_ref[...]
    )


def _linear(x, wt, b2d, block_rows):
    n, cin = x.shape
    cout = wt.shape[1]
    return pl.pallas_call(
        _linear_body,
        grid=(n // block_rows,),
        in_specs=[
            pl.BlockSpec((block_rows, cin), lambda i: (i, 0)),
            pl.BlockSpec((cin, cout), lambda i: (0, 0)),
            pl.BlockSpec((1, cout), lambda i: (0, 0)),
        ],
        out_specs=pl.BlockSpec((block_rows, cout), lambda i: (i, 0)),
        out_shape=jax.ShapeDtypeStruct((n, cout), jnp.float32),
    )(x, wt, b2d)


def _combine_body(tr_ref, agg_ref, cnt_ref, tbl_ref, o_ref):
    c = cnt_ref[0] + cnt_ref[1]
    emb = jnp.dot(c, tbl_ref[...], preferred_element_type=jnp.float32)
    o_ref[...] = jnp.maximum(agg_ref[0] + agg_ref[1] + tr_ref[...] + emb, 0.0)


def _combine(tr, agg, cnt3, tbl, block_rows):
    n, cout = tr.shape
    n_types = tbl.shape[0]
    return pl.pallas_call(
        _combine_body,
        grid=(n // block_rows,),
        in_specs=[
            pl.BlockSpec((block_rows, cout), lambda i: (i, 0)),
            pl.BlockSpec((2, block_rows, cout), lambda i: (0, i, 0)),
            pl.BlockSpec((2, block_rows, n_types), lambda i: (0, i, 0)),
            pl.BlockSpec((n_types, cout), lambda i: (0, 0)),
        ],
        out_specs=pl.BlockSpec((block_rows, cout), lambda i: (i, 0)),
        out_shape=jax.ShapeDtypeStruct((n, cout), jnp.float32),
    )(tr, agg, cnt3, tbl)


def _mesh():
    return plsc.VectorSubcoreMesh(core_axis_name="c", subcore_axis_name="s")


@functools.lru_cache(maxsize=None)
def _make_sc_agg(npad, nchunks, cout):
    rows_per_tile = npad // 16

    @functools.partial(
        pl.kernel,
        out_type=jax.ShapeDtypeStruct((2, npad, cout), jnp.float32),
        mesh=_mesh(),
        scratch_types=[
            pltpu.VMEM((nchunks, _CHUNK), jnp.int32),
            pltpu.VMEM((nchunks, _CHUNK), jnp.int32),
            pltpu.VMEM((_CHUNK, 128), jnp.float32),
            pltpu.VMEM_SHARED((npad, 128), jnp.float32),
            pltpu.SemaphoreType.DMA,
        ],
    )
    def sc_agg(tr_hbm, src_hbm, tgt_hbm, agg_out,
               src_v, tgt_v, rows_v, agg_sh, gsem):
        cid = lax.axis_index("c")
        sid = lax.axis_index("s")
        wid = cid * 16 + sid

        zero16 = jnp.zeros((_LANES,), jnp.float32)

        def _zr(r, carry):
            for i in range(cout // _LANES):
                rows_v[r, pl.ds(i * _LANES, _LANES)] = zero16
            return carry

        lax.fori_loop(0, _CHUNK, _zr, 0)

        rbase = sid * rows_per_tile
        nfull, rrem = divmod(rows_per_tile, _CHUNK)
        for k in range(nfull):
            pltpu.sync_copy(rows_v, agg_sh.at[pl.ds(rbase + k * _CHUNK, _CHUNK)])
        if rrem:
            pltpu.sync_copy(
                rows_v.at[pl.ds(0, rrem)],
                agg_sh.at[pl.ds(rbase + nfull * _CHUNK, rrem)],
            )

        pltpu.sync_copy(src_hbm.at[pl.ds(wid * nchunks, nchunks)], src_v)
        pltpu.sync_copy(tgt_hbm.at[pl.ds(wid * nchunks, nchunks)], tgt_v)

        plsc.subcore_barrier()

        def _chunk(j, carry):
            pltpu.async_copy(tr_hbm.at[src_v.at[j]], rows_v, gsem).wait()
            pltpu.sync_copy(rows_v, agg_sh.at[tgt_v.at[j]], add=True)
            return carry

        lax.fori_loop(0, nchunks, _chunk, 0)

        plsc.subcore_barrier()

        pltpu.sync_copy(
            agg_sh.at[pl.ds(rbase, rows_per_tile)],
            agg_out.at[cid, pl.ds(rbase, rows_per_tile)],
        )

    return sc_agg


@functools.lru_cache(maxsize=None)
def _make_sc_cnt(npad, nchunks, n_types):
    cnt_len = npad * n_types
    cnt_per_tile = cnt_len // 16

    @functools.partial(
        pl.kernel,
        out_type=jax.ShapeDtypeStruct((2, cnt_len), jnp.float32),
        mesh=_mesh(),
        scratch_types=[
            pltpu.VMEM((nchunks, _CHUNK), jnp.int32),
            pltpu.VMEM((nchunks, _CHUNK), jnp.int32),
            pltpu.VMEM((_CHUNK,), jnp.int32),
            pltpu.VMEM((_CHUNK,), jnp.float32),
            pltpu.VMEM((2048,), jnp.float32),
            pltpu.VMEM_SHARED((cnt_len,), jnp.float32),
        ],
    )
    def sc_cnt(tgt_hbm, et_hbm, cnt_out,
               tgt_v, et_v, cidx_v, ones_v, zf_v, cnt_sh):
        cid = lax.axis_index("c")
        sid = lax.axis_index("s")
        wid = cid * 16 + sid

        zero16 = jnp.zeros((_LANES,), jnp.float32)
        one16 = jnp.ones((_LANES,), jnp.float32)

        def _zf(k, carry):
            zf_v[pl.ds(k * _LANES, _LANES)] = zero16
            return carry

        lax.fori_loop(0, 2048 // _LANES, _zf, 0)

        for i in range(_CHUNK // _LANES):
            ones_v[pl.ds(i * _LANES, _LANES)] = one16

        cbase = sid * cnt_per_tile
        cfull, crem = divmod(cnt_per_tile, 2048)
        for k in range(cfull):
            pltpu.sync_copy(zf_v, cnt_sh.at[pl.ds(cbase + k * 2048, 2048)])
        if crem:
            pltpu.sync_copy(
                zf_v.at[pl.ds(0, crem)],
                cnt_sh.at[pl.ds(cbase + cfull * 2048, crem)],
            )

        pltpu.sync_copy(tgt_hbm.at[pl.ds(wid * nchunks, nchunks)], tgt_v)
        pltpu.sync_copy(et_hbm.at[pl.ds(wid * nchunks, nchunks)], et_v)

        plsc.subcore_barrier()

        def _chunk(j, carry):
            for i in range(_CHUNK // _LANES):
                sl = pl.ds(i * _LANES, _LANES)
                cidx_v[sl] = tgt_v[j, sl] * n_types + et_v[j, sl]
            pltpu.sync_copy(ones_v, cnt_sh.at[cidx_v], add=True)
            return carry

        lax.fori_loop(0, nchunks, _chunk, 0)

        plsc.subcore_barrier()

        pltpu.sync_copy(
            cnt_sh.at[pl.ds(cbase, cnt_per_tile)],
            cnt_out.at[cid, pl.ds(cbase, cnt_per_tile)],
        )

    return sc_cnt


def kernel(node_features, edge_index, edge_type, W, b, edge_emb_table):
    n, cin = node_features.shape
    cout, _ = W.shape
    n_types = edge_emb_table.shape[0]
    e = edge_index.shape[1]

    src = edge_index[0].astype(jnp.int32)
    tgt = edge_index[1].astype(jnp.int32)
    et = edge_type.astype(jnp.int32)

    npad = -(-(n + 1) // 128) * 128
    ept = -(-e // (_NUM_WORKERS * _CHUNK * 8)) * (_CHUNK * 8)
    epad = ept * _NUM_WORKERS
    pad = epad - e
    if pad:
        src = jnp.concatenate([src, jnp.zeros((pad,), jnp.int32)])
        tgt = jnp.concatenate([tgt, jnp.full((pad,), n, jnp.int32)])
        et = jnp.concatenate([et, jnp.zeros((pad,), jnp.int32)])
    src2 = src.reshape(epad // _CHUNK, _CHUNK)
    tgt2 = tgt.reshape(epad // _CHUNK, _CHUNK)
    et2 = et.reshape(epad // _CHUNK, _CHUNK)

    transformed = _linear(node_features, W.T, b.reshape(1, cout), block_rows=2000)

    nchunks = ept // _CHUNK
    agg = _make_sc_agg(npad, nchunks, cout)(transformed, src2, tgt2)
    cnt = _make_sc_cnt(npad, nchunks, n_types)(tgt2, et2)

    cnt3 = cnt.reshape(2, npad, n_types)
    return _combine(transformed, agg, cnt3, edge_emb_table, block_rows=2000)


# speedup vs baseline: 5.9562x; 1.2917x over previous
import functools

import jax
import jax.numpy as jnp
from jax import lax
from jax.experimental import pallas as pl
from jax.experimental.pallas import tpu as pltpu
from jax.experimental.pallas import tpu_sc as plsc

_LANES = 16
_CHUNK = 128
_NUM_WORKERS = 32
_NBUF = 2
_NODE_BITS = 14


def _linear_body(x_ref, wt_ref, b_ref, o_ref):
    o_ref[...] = (
        jnp.dot(x_ref[...], wt_ref[...], preferred_element_type=jnp.float32)
        + b_ref[...]
    )


def _linear(x, wt, b2d, block_rows):
    n, cin = x.shape
    cout = wt.shape[1]
    return pl.pallas_call(
        _linear_body,
        grid=(n // block_rows,),
        in_specs=[
            pl.BlockSpec((block_rows, cin), lambda i: (i, 0)),
            pl.BlockSpec((cin, cout), lambda i: (0, 0)),
            pl.BlockSpec((1, cout), lambda i: (0, 0)),
        ],
        out_specs=pl.BlockSpec((block_rows, cout), lambda i: (i, 0)),
        out_shape=jax.ShapeDtypeStruct((n, cout), jnp.float32),
    )(x, wt, b2d)


def _combine_body(tr_ref, agg_ref, cnt_ref, tbl_ref, o_ref):
    c = cnt_ref[0] + cnt_ref[1]
    emb = jnp.dot(c, tbl_ref[...], preferred_element_type=jnp.float32)
    o_ref[...] = jnp.maximum(agg_ref[0] + agg_ref[1] + tr_ref[...] + emb, 0.0)


def _combine(tr, agg, cnt3, tbl, block_rows):
    n, cout = tr.shape
    n_types = tbl.shape[0]
    return pl.pallas_call(
        _combine_body,
        grid=(n // block_rows,),
        in_specs=[
            pl.BlockSpec((block_rows, cout), lambda i: (i, 0)),
            pl.BlockSpec((2, block_rows, cout), lambda i: (0, i, 0)),
            pl.BlockSpec((2, block_rows, n_types), lambda i: (0, i, 0)),
            pl.BlockSpec((n_types, cout), lambda i: (0, 0)),
        ],
        out_specs=pl.BlockSpec((block_rows, cout), lambda i: (i, 0)),
        out_shape=jax.ShapeDtypeStruct((n, cout), jnp.float32),
    )(tr, agg, cnt3, tbl)


def _mesh():
    return plsc.VectorSubcoreMesh(core_axis_name="c", subcore_axis_name="s")


@functools.lru_cache(maxsize=None)
def _make_sc_agg(npad, nchunks, cout):
    rows_per_tile = npad // 16
    ngroups = nchunks // _NBUF
    nmask = (1 << _NODE_BITS) - 1

    @functools.partial(
        pl.kernel,
        out_type=jax.ShapeDtypeStruct((2, npad, cout), jnp.float32),
        mesh=_mesh(),
        scratch_types=[
            pltpu.VMEM((nchunks, _CHUNK), jnp.int32),
            pltpu.VMEM((_NBUF, _CHUNK), jnp.int32),
            pltpu.VMEM((_NBUF, _CHUNK), jnp.int32),
            pltpu.VMEM((_NBUF, _CHUNK, 128), jnp.float32),
            pltpu.VMEM_SHARED((npad, 128), jnp.float32),
        ]
        + [pltpu.SemaphoreType.DMA] * (2 * _NBUF),
    )
    def sc_agg(tr_hbm, pk_hbm, agg_out,
               pk_v, src_v, tgt_v, rows_v, agg_sh, *sems):
        gsem = sems[:_NBUF]
        ssem = sems[_NBUF:]
        cid = lax.axis_index("c")
        sid = lax.axis_index("s")
        wid = cid * 16 + sid

        zero16 = jnp.zeros((_LANES,), jnp.float32)

        pltpu.sync_copy(pk_hbm.at[pl.ds(wid * nchunks, nchunks)], pk_v)

        def _dec_src(j, b):
            for i in range(_CHUNK // _LANES):
                sl = pl.ds(i * _LANES, _LANES)
                src_v[b, sl] = pk_v[j, sl] & nmask

        def _dec_tgt(j, b):
            for i in range(_CHUNK // _LANES):
                sl = pl.ds(i * _LANES, _LANES)
                tgt_v[b, sl] = (
                    lax.shift_right_logical(pk_v[j, sl], _NODE_BITS) & nmask
                )

        def _zr(r, carry):
            for i in range(cout // _LANES):
                rows_v[0, r, pl.ds(i * _LANES, _LANES)] = zero16
            return carry

        lax.fori_loop(0, _CHUNK, _zr, 0)

        rbase = sid * rows_per_tile
        zsrc = rows_v.at[0]
        nfull, rrem = divmod(rows_per_tile, _CHUNK)
        for k in range(nfull):
            pltpu.sync_copy(zsrc, agg_sh.at[pl.ds(rbase + k * _CHUNK, _CHUNK)])
        if rrem:
            pltpu.sync_copy(
                zsrc.at[pl.ds(0, rrem)],
                agg_sh.at[pl.ds(rbase + nfull * _CHUNK, rrem)],
            )

        plsc.subcore_barrier()

        def _fire_gather(b):
            pltpu.async_copy(tr_hbm.at[src_v.at[b]], rows_v.at[b], gsem[b])

        def _wait_gather(b):
            pltpu.make_async_copy(
                tr_hbm.at[src_v.at[b]], rows_v.at[b], gsem[b]
            ).wait()

        def _fire_scatter(b):
            pltpu.async_copy(
                rows_v.at[b], agg_sh.at[tgt_v.at[b]], ssem[b], add=True
            )

        def _wait_scatter(b):
            pltpu.make_async_copy(
                rows_v.at[b], agg_sh.at[tgt_v.at[b]], ssem[b]
            ).wait()

        for b in range(_NBUF):
            _dec_src(b, b)
            _fire_gather(b)

        def _group(g, carry):
            base = g * _NBUF
            for b in range(_NBUF):
                _wait_gather(b)
                _dec_tgt(base + b, b)
                _fire_scatter(b)
            for b in range(_NBUF):
                _wait_scatter(b)
                _dec_src(base + _NBUF + b, b)
                _fire_gather(b)
            return carry

        lax.fori_loop(0, ngroups - 1, _group, 0)

        tail = (ngroups - 1) * _NBUF
        for b in range(_NBUF):
            _wait_gather(b)
            _dec_tgt(tail + b, b)
            _fire_scatter(b)
        for b in range(_NBUF):
            _wait_scatter(b)

        plsc.subcore_barrier()

        pltpu.sync_copy(
            agg_sh.at[pl.ds(rbase, rows_per_tile)],
            agg_out.at[cid, pl.ds(rbase, rows_per_tile)],
        )

    return sc_agg


@functools.lru_cache(maxsize=None)
def _make_sc_cnt(npad, nchunks, n_types):
    cnt_len = npad * n_types
    cnt_per_tile = cnt_len // 16
    nmask = (1 << _NODE_BITS) - 1
    tmask = n_types - 1

    @functools.partial(
        pl.kernel,
        out_type=jax.ShapeDtypeStruct((2, cnt_len), jnp.float32),
        mesh=_mesh(),
        scratch_types=[
            pltpu.VMEM((nchunks, _CHUNK), jnp.int32),
            pltpu.VMEM((_CHUNK,), jnp.int32),
            pltpu.VMEM((_CHUNK,), jnp.float32),
            pltpu.VMEM((2048,), jnp.float32),
            pltpu.VMEM_SHARED((cnt_len,), jnp.float32),
        ],
    )
    def sc_cnt(pk_hbm, cnt_out, pk_v, cidx_v, ones_v, zf_v, cnt_sh):
        cid = lax.axis_index("c")
        sid = lax.axis_index("s")
        wid = cid * 16 + sid

        zero16 = jnp.zeros((_LANES,), jnp.float32)
        one16 = jnp.ones((_LANES,), jnp.float32)

        def _zf(k, carry):
            zf_v[pl.ds(k * _LANES, _LANES)] = zero16
            return carry

        lax.fori_loop(0, 2048 // _LANES, _zf, 0)

        for i in range(_CHUNK // _LANES):
            ones_v[pl.ds(i * _LANES, _LANES)] = one16

        cbase = sid * cnt_per_tile
        cfull, crem = divmod(cnt_per_tile, 2048)
        for k in range(cfull):
            pltpu.sync_copy(zf_v, cnt_sh.at[pl.ds(cbase + k * 2048, 2048)])
        if crem:
            pltpu.sync_copy(
                zf_v.at[pl.ds(0, crem)],
                cnt_sh.at[pl.ds(cbase + cfull * 2048, crem)],
            )

        pltpu.sync_copy(pk_hbm.at[pl.ds(wid * nchunks, nchunks)], pk_v)

        plsc.subcore_barrier()

        def _chunk(j, carry):
            for i in range(_CHUNK // _LANES):
                sl = pl.ds(i * _LANES, _LANES)
                p = pk_v[j, sl]
                tgt = lax.shift_right_logical(p, _NODE_BITS) & nmask
                et = lax.shift_right_logical(p, 2 * _NODE_BITS) & tmask
                cidx_v[sl] = tgt * n_types + et
            pltpu.sync_copy(ones_v, cnt_sh.at[cidx_v], add=True)
            return carry

        lax.fori_loop(0, nchunks, _chunk, 0)

        plsc.subcore_barrier()

        pltpu.sync_copy(
            cnt_sh.at[pl.ds(cbase, cnt_per_tile)],
            cnt_out.at[cid, pl.ds(cbase, cnt_per_tile)],
        )

    return sc_cnt


def kernel(node_features, edge_index, edge_type, W, b, edge_emb_table):
    n, cin = node_features.shape
    cout, _ = W.shape
    n_types = edge_emb_table.shape[0]
    e = edge_index.shape[1]
    assert n + 1 <= (1 << _NODE_BITS) and n_types <= (1 << (32 - 2 * _NODE_BITS))

    src = edge_index[0].astype(jnp.int32)
    tgt = edge_index[1].astype(jnp.int32)
    et = edge_type.astype(jnp.int32)

    npad = -(-(n + 1) // 128) * 128
    ept = -(-e // (_NUM_WORKERS * _CHUNK * 8)) * (_CHUNK * 8)
    epad = ept * _NUM_WORKERS
    pad = epad - e
    if pad:
        src = jnp.concatenate([src, jnp.zeros((pad,), jnp.int32)])
        tgt = jnp.concatenate([tgt, jnp.full((pad,), n, jnp.int32)])
        et = jnp.concatenate([et, jnp.zeros((pad,), jnp.int32)])
    packed = src | (tgt << _NODE_BITS) | (et << (2 * _NODE_BITS))
    pk2 = packed.reshape(epad // _CHUNK, _CHUNK)

    transformed = _linear(node_features, W.T, b.reshape(1, cout), block_rows=2000)

    nchunks = ept // _CHUNK
    agg = _make_sc_agg(npad, nchunks, cout)(transformed, pk2)
    cnt = _make_sc_cnt(npad, nchunks, n_types)(pk2)

    cnt3 = cnt.reshape(2, npad, n_types)
    return _combine(transformed, agg, cnt3, edge_emb_table, block_rows=2000)


# speedup vs baseline: 6.0826x; 1.0212x over previous
import functools

import jax
import jax.numpy as jnp
from jax import lax
from jax.experimental import pallas as pl
from jax.experimental.pallas import tpu as pltpu
from jax.experimental.pallas import tpu_sc as plsc

_LANES = 16
_CHUNK = 64
_PKW = 128
_NUM_WORKERS = 32
_NBUF = 4
_NODE_BITS = 14


def _linear_body(x_ref, wt_ref, b_ref, o_ref):
    o_ref[...] = (
        jnp.dot(x_ref[...], wt_ref[...], preferred_element_type=jnp.float32)
        + b_ref[...]
    )


def _linear(x, wt, b2d, block_rows):
    n, cin = x.shape
    cout = wt.shape[1]
    return pl.pallas_call(
        _linear_body,
        grid=(n // block_rows,),
        in_specs=[
            pl.BlockSpec((block_rows, cin), lambda i: (i, 0)),
            pl.BlockSpec((cin, cout), lambda i: (0, 0)),
            pl.BlockSpec((1, cout), lambda i: (0, 0)),
        ],
        out_specs=pl.BlockSpec((block_rows, cout), lambda i: (i, 0)),
        out_shape=jax.ShapeDtypeStruct((n, cout), jnp.float32),
    )(x, wt, b2d)


def _combine_body(tr_ref, agg_ref, cnt_ref, tbl_ref, o_ref):
    c = cnt_ref[0] + cnt_ref[1]
    emb = jnp.dot(c, tbl_ref[...], preferred_element_type=jnp.float32)
    o_ref[...] = jnp.maximum(agg_ref[0] + agg_ref[1] + tr_ref[...] + emb, 0.0)


def _combine(tr, agg, cnt3, tbl, block_rows):
    n, cout = tr.shape
    n_types = tbl.shape[0]
    return pl.pallas_call(
        _combine_body,
        grid=(n // block_rows,),
        in_specs=[
            pl.BlockSpec((block_rows, cout), lambda i: (i, 0)),
            pl.BlockSpec((2, block_rows, cout), lambda i: (0, i, 0)),
            pl.BlockSpec((2, block_rows, n_types), lambda i: (0, i, 0)),
            pl.BlockSpec((n_types, cout), lambda i: (0, 0)),
        ],
        out_specs=pl.BlockSpec((block_rows, cout), lambda i: (i, 0)),
        out_shape=jax.ShapeDtypeStruct((n, cout), jnp.float32),
    )(tr, agg, cnt3, tbl)


def _mesh():
    return plsc.VectorSubcoreMesh(core_axis_name="c", subcore_axis_name="s")


@functools.lru_cache(maxsize=None)
def _make_sc_agg(npad, nchunks, cout):
    rows_per_tile = npad // 16
    ngroups = nchunks // _NBUF
    nmask = (1 << _NODE_BITS) - 1

    @functools.partial(
        pl.kernel,
        out_type=jax.ShapeDtypeStruct((2, npad, cout), jnp.float32),
        mesh=_mesh(),
        scratch_types=[
            pltpu.VMEM((nchunks // 2, _PKW), jnp.int32),
            pltpu.VMEM((_NBUF, _CHUNK), jnp.int32),
            pltpu.VMEM((_NBUF, _CHUNK), jnp.int32),
            pltpu.VMEM((_NBUF, _CHUNK, 128), jnp.float32),
            pltpu.VMEM_SHARED((npad, 128), jnp.float32),
        ]
        + [pltpu.SemaphoreType.DMA] * (2 * _NBUF),
    )
    def sc_agg(tr_hbm, pk_hbm, agg_out,
               pk_v, src_v, tgt_v, rows_v, agg_sh, *sems):
        gsem = sems[:_NBUF]
        ssem = sems[_NBUF:]
        cid = lax.axis_index("c")
        sid = lax.axis_index("s")
        wid = cid * 16 + sid

        zero16 = jnp.zeros((_LANES,), jnp.float32)

        npkrows = nchunks // 2
        pltpu.sync_copy(pk_hbm.at[pl.ds(wid * npkrows, npkrows)], pk_v)

        def _dec_src(c, b):
            half = (b % 2) * _CHUNK
            for i in range(_CHUNK // _LANES):
                pv = pk_v[c // 2, pl.ds(half + i * _LANES, _LANES)]
                src_v[b, pl.ds(i * _LANES, _LANES)] = pv & nmask

        def _dec_tgt(c, b):
            half = (b % 2) * _CHUNK
            for i in range(_CHUNK // _LANES):
                pv = pk_v[c // 2, pl.ds(half + i * _LANES, _LANES)]
                tgt_v[b, pl.ds(i * _LANES, _LANES)] = (
                    lax.shift_right_logical(pv, _NODE_BITS) & nmask
                )

        def _zr(r, carry):
            for i in range(cout // _LANES):
                rows_v[0, r, pl.ds(i * _LANES, _LANES)] = zero16
            return carry

        lax.fori_loop(0, _CHUNK, _zr, 0)

        rbase = sid * rows_per_tile
        zsrc = rows_v.at[0]
        nfull, rrem = divmod(rows_per_tile, _CHUNK)
        for k in range(nfull):
            pltpu.sync_copy(zsrc, agg_sh.at[pl.ds(rbase + k * _CHUNK, _CHUNK)])
        if rrem:
            pltpu.sync_copy(
                zsrc.at[pl.ds(0, rrem)],
                agg_sh.at[pl.ds(rbase + nfull * _CHUNK, rrem)],
            )

        plsc.subcore_barrier()

        def _fire_gather(b):
            pltpu.async_copy(tr_hbm.at[src_v.at[b]], rows_v.at[b], gsem[b])

        def _wait_gather(b):
            pltpu.make_async_copy(
                tr_hbm.at[src_v.at[b]], rows_v.at[b], gsem[b]
            ).wait()

        def _fire_scatter(b):
            pltpu.async_copy(
                rows_v.at[b], agg_sh.at[tgt_v.at[b]], ssem[b], add=True
            )

        def _wait_scatter(b):
            pltpu.make_async_copy(
                rows_v.at[b], agg_sh.at[tgt_v.at[b]], ssem[b]
            ).wait()

        for b in range(_NBUF):
            _dec_src(b, b)
            _fire_gather(b)

        def _group(g, carry):
            base = g * _NBUF
            for b in range(_NBUF):
                _wait_gather(b)
                _dec_tgt(base + b, b)
                _fire_scatter(b)
            for b in range(_NBUF):
                _wait_scatter(b)
                _dec_src(base + _NBUF + b, b)
                _fire_gather(b)
            return carry

        lax.fori_loop(0, ngroups - 1, _group, 0)

        tail = (ngroups - 1) * _NBUF
        for b in range(_NBUF):
            _wait_gather(b)
            _dec_tgt(tail + b, b)
            _fire_scatter(b)
        for b in range(_NBUF):
            _wait_scatter(b)

        plsc.subcore_barrier()

        pltpu.sync_copy(
            agg_sh.at[pl.ds(rbase, rows_per_tile)],
            agg_out.at[cid, pl.ds(rbase, rows_per_tile)],
        )

    return sc_agg


@functools.lru_cache(maxsize=None)
def _make_sc_cnt(npad, nchunks, n_types):
    cnt_len = npad * n_types
    cnt_per_tile = cnt_len // 16
    nmask = (1 << _NODE_BITS) - 1
    tmask = n_types - 1

    @functools.partial(
        pl.kernel,
        out_type=jax.ShapeDtypeStruct((2, cnt_len), jnp.float32),
        mesh=_mesh(),
        scratch_types=[
            pltpu.VMEM((nchunks, _PKW), jnp.int32),
            pltpu.VMEM((_PKW,), jnp.int32),
            pltpu.VMEM((_PKW,), jnp.float32),
            pltpu.VMEM((2048,), jnp.float32),
            pltpu.VMEM_SHARED((cnt_len,), jnp.float32),
        ],
    )
    def sc_cnt(pk_hbm, cnt_out, pk_v, cidx_v, ones_v, zf_v, cnt_sh):
        cid = lax.axis_index("c")
        sid = lax.axis_index("s")
        wid = cid * 16 + sid

        zero16 = jnp.zeros((_LANES,), jnp.float32)
        one16 = jnp.ones((_LANES,), jnp.float32)

        def _zf(k, carry):
            zf_v[pl.ds(k * _LANES, _LANES)] = zero16
            return carry

        lax.fori_loop(0, 2048 // _LANES, _zf, 0)

        for i in range(_PKW // _LANES):
            ones_v[pl.ds(i * _LANES, _LANES)] = one16

        cbase = sid * cnt_per_tile
        cfull, crem = divmod(cnt_per_tile, 2048)
        for k in range(cfull):
            pltpu.sync_copy(zf_v, cnt_sh.at[pl.ds(cbase + k * 2048, 2048)])
        if crem:
            pltpu.sync_copy(
                zf_v.at[pl.ds(0, crem)],
                cnt_sh.at[pl.ds(cbase + cfull * 2048, crem)],
            )

        pltpu.sync_copy(pk_hbm.at[pl.ds(wid * nchunks, nchunks)], pk_v)

        plsc.subcore_barrier()

        def _chunk(j, carry):
            for i in range(_PKW // _LANES):
                sl = pl.ds(i * _LANES, _LANES)
                p = pk_v[j, sl]
                tgt = lax.shift_right_logical(p, _NODE_BITS) & nmask
                et = lax.shift_right_logical(p, 2 * _NODE_BITS) & tmask
                cidx_v[sl] = tgt * n_types + et
            pltpu.sync_copy(ones_v, cnt_sh.at[cidx_v], add=True)
            return carry

        lax.fori_loop(0, nchunks, _chunk, 0)

        plsc.subcore_barrier()

        pltpu.sync_copy(
            cnt_sh.at[pl.ds(cbase, cnt_per_tile)],
            cnt_out.at[cid, pl.ds(cbase, cnt_per_tile)],
        )

    return sc_cnt


def kernel(node_features, edge_index, edge_type, W, b, edge_emb_table):
    n, cin = node_features.shape
    cout, _ = W.shape
    n_types = edge_emb_table.shape[0]
    e = edge_index.shape[1]
    assert n + 1 <= (1 << _NODE_BITS) and n_types <= (1 << (32 - 2 * _NODE_BITS))

    src = edge_index[0].astype(jnp.int32)
    tgt = edge_index[1].astype(jnp.int32)
    et = edge_type.astype(jnp.int32)

    npad = -(-(n + 1) // 128) * 128
    ept = -(-e // (_NUM_WORKERS * _PKW * 8)) * (_PKW * 8)
    epad = ept * _NUM_WORKERS
    pad = epad - e
    if pad:
        src = jnp.concatenate([src, jnp.zeros((pad,), jnp.int32)])
        tgt = jnp.concatenate([tgt, jnp.full((pad,), n, jnp.int32)])
        et = jnp.concatenate([et, jnp.zeros((pad,), jnp.int32)])
    packed = src | (tgt << _NODE_BITS) | (et << (2 * _NODE_BITS))
    pk2 = packed.reshape(epad // _PKW, _PKW)

    transformed = _linear(node_features, W.T, b.reshape(1, cout), block_rows=2000)

    agg = _make_sc_agg(npad, ept // _CHUNK, cout)(transformed, pk2)
    cnt = _make_sc_cnt(npad, epad // _PKW // _NUM_WORKERS, n_types)(pk2)

    cnt3 = cnt.reshape(2, npad, n_types)
    return _combine(transformed, agg, cnt3, edge_emb_table, block_rows=2000)


# speedup vs baseline: 15.8160x; 2.6002x over previous
import functools

import jax
import jax.numpy as jnp
from jax import lax
from jax.experimental import pallas as pl
from jax.experimental.pallas import tpu as pltpu
from jax.experimental.pallas import tpu_sc as plsc

_LANES = 16
_CHUNK = 64
_PKW = 128
_NUM_WORKERS = 32
_NBUF = 4
_NODE_BITS = 14


def _linear_body(x_ref, wt_ref, b_ref, o_ref):
    o_ref[...] = (
        jnp.dot(x_ref[...], wt_ref[...], preferred_element_type=jnp.float32)
        + b_ref[...]
    )


def _linear(x, wt, b2d, block_rows):
    n, cin = x.shape
    cout = wt.shape[1]
    return pl.pallas_call(
        _linear_body,
        grid=(n // block_rows,),
        in_specs=[
            pl.BlockSpec((block_rows, cin), lambda i: (i, 0)),
            pl.BlockSpec((cin, cout), lambda i: (0, 0)),
            pl.BlockSpec((1, cout), lambda i: (0, 0)),
        ],
        out_specs=pl.BlockSpec((block_rows, cout), lambda i: (i, 0)),
        out_shape=jax.ShapeDtypeStruct((n, cout), jnp.float32),
    )(x, wt, b2d)


def _combine_body(tr_ref, agg_ref, cnt_ref, tbl_ref, o_ref):
    c = cnt_ref[0] + cnt_ref[1]
    emb = jnp.dot(c, tbl_ref[...], preferred_element_type=jnp.float32)
    o_ref[...] = jnp.maximum(agg_ref[0] + agg_ref[1] + tr_ref[...] + emb, 0.0)


def _combine(tr, agg, cnt3, tbl, block_rows):
    n, cout = tr.shape
    n_types = tbl.shape[0]
    return pl.pallas_call(
        _combine_body,
        grid=(n // block_rows,),
        in_specs=[
            pl.BlockSpec((block_rows, cout), lambda i: (i, 0)),
            pl.BlockSpec((2, block_rows, cout), lambda i: (0, i, 0)),
            pl.BlockSpec((2, block_rows, n_types), lambda i: (0, i, 0)),
            pl.BlockSpec((n_types, cout), lambda i: (0, 0)),
        ],
        out_specs=pl.BlockSpec((block_rows, cout), lambda i: (i, 0)),
        out_shape=jax.ShapeDtypeStruct((n, cout), jnp.float32),
    )(tr, agg, cnt3, tbl)


def _mesh():
    return plsc.VectorSubcoreMesh(core_axis_name="c", subcore_axis_name="s")


@functools.lru_cache(maxsize=None)
def _make_sc_agg(npad, nchunks, cout):
    rows_per_tile = npad // 16
    ngroups = nchunks // _NBUF
    nmask = (1 << _NODE_BITS) - 1

    @functools.partial(
        pl.kernel,
        out_type=jax.ShapeDtypeStruct((2, npad, cout), jnp.float32),
        mesh=_mesh(),
        scratch_types=[
            pltpu.VMEM((nchunks // 2, _PKW), jnp.int32),
            pltpu.VMEM((_NBUF, _CHUNK), jnp.int32),
            pltpu.VMEM((_NBUF, _CHUNK), jnp.int32),
            pltpu.VMEM((_NBUF, _CHUNK, 128), jnp.float32),
            pltpu.VMEM_SHARED((npad, 128), jnp.float32),
        ]
        + [pltpu.SemaphoreType.DMA] * (2 * _NBUF),
    )
    def sc_agg(tr_hbm, pk_hbm, agg_out,
               pk_v, src_v, tgt_v, rows_v, agg_sh, *sems):
        gsem = sems[:_NBUF]
        ssem = sems[_NBUF:]
        cid = lax.axis_index("c")
        sid = lax.axis_index("s")
        wid = cid * 16 + sid

        zero16 = jnp.zeros((_LANES,), jnp.float32)

        npkrows = nchunks // 2
        pltpu.sync_copy(pk_hbm.at[pl.ds(wid * npkrows, npkrows)], pk_v)

        def _dec_src(c, b):
            half = (b % 2) * _CHUNK
            for i in range(_CHUNK // _LANES):
                pv = pk_v[c // 2, pl.ds(half + i * _LANES, _LANES)]
                src_v[b, pl.ds(i * _LANES, _LANES)] = pv & nmask

        def _dec_tgt(c, b):
            half = (b % 2) * _CHUNK
            for i in range(_CHUNK // _LANES):
                pv = pk_v[c // 2, pl.ds(half + i * _LANES, _LANES)]
                tgt_v[b, pl.ds(i * _LANES, _LANES)] = (
                    lax.shift_right_logical(pv, _NODE_BITS) & nmask
                )

        def _zr(r, carry):
            for i in range(cout // _LANES):
                rows_v[0, r, pl.ds(i * _LANES, _LANES)] = zero16
            return carry

        lax.fori_loop(0, _CHUNK, _zr, 0)

        rbase = sid * rows_per_tile
        zsrc = rows_v.at[0]
        nfull, rrem = divmod(rows_per_tile, _CHUNK)
        for k in range(nfull):
            pltpu.sync_copy(zsrc, agg_sh.at[pl.ds(rbase + k * _CHUNK, _CHUNK)])
        if rrem:
            pltpu.sync_copy(
                zsrc.at[pl.ds(0, rrem)],
                agg_sh.at[pl.ds(rbase + nfull * _CHUNK, rrem)],
            )

        plsc.subcore_barrier()

        def _fire_gather(b):
            pltpu.async_copy(tr_hbm.at[src_v.at[b]], rows_v.at[b], gsem[b])

        def _wait_gather(b):
            pltpu.make_async_copy(
                tr_hbm.at[src_v.at[b]], rows_v.at[b], gsem[b]
            ).wait()

        def _fire_scatter(b):
            pltpu.async_copy(
                rows_v.at[b], agg_sh.at[tgt_v.at[b]], ssem[b], add=True
            )

        def _wait_scatter(b):
            pltpu.make_async_copy(
                rows_v.at[b], agg_sh.at[tgt_v.at[b]], ssem[b]
            ).wait()

        for b in range(_NBUF):
            _dec_src(b, b)
            _fire_gather(b)

        def _group(g, carry):
            base = g * _NBUF
            for b in range(_NBUF):
                _wait_gather(b)
                _dec_tgt(base + b, b)
                _fire_scatter(b)
            for b in range(_NBUF):
                _wait_scatter(b)
                _dec_src(base + _NBUF + b, b)
                _fire_gather(b)
            return carry

        lax.fori_loop(0, ngroups - 1, _group, 0)

        tail = (ngroups - 1) * _NBUF
        for b in range(_NBUF):
            _wait_gather(b)
            _dec_tgt(tail + b, b)
            _fire_scatter(b)
        for b in range(_NBUF):
            _wait_scatter(b)

        plsc.subcore_barrier()

        pltpu.sync_copy(
            agg_sh.at[pl.ds(rbase, rows_per_tile)],
            agg_out.at[cid, pl.ds(rbase, rows_per_tile)],
        )

    return sc_agg


@functools.lru_cache(maxsize=None)
def _make_sc_cnt(npad, nchunks, n_types):
    cnt_len = npad * n_types
    cnt_per_tile = cnt_len // 16
    nmask = (1 << _NODE_BITS) - 1
    tmask = n_types - 1

    @functools.partial(
        pl.kernel,
        out_type=jax.ShapeDtypeStruct((2, cnt_len), jnp.float32),
        mesh=_mesh(),
        scratch_types=[
            pltpu.VMEM((nchunks, _PKW), jnp.int32),
            pltpu.VMEM((_PKW,), jnp.int32),
            pltpu.VMEM((_PKW,), jnp.float32),
            pltpu.VMEM((2048,), jnp.float32),
            pltpu.VMEM_SHARED((cnt_len,), jnp.float32),
        ],
    )
    def sc_cnt(pk_hbm, cnt_out, pk_v, cidx_v, ones_v, zf_v, cnt_sh):
        cid = lax.axis_index("c")
        sid = lax.axis_index("s")
        wid = cid * 16 + sid

        zero16 = jnp.zeros((_LANES,), jnp.float32)
        one16 = jnp.ones((_LANES,), jnp.float32)

        def _zf(k, carry):
            zf_v[pl.ds(k * _LANES, _LANES)] = zero16
            return carry

        lax.fori_loop(0, 2048 // _LANES, _zf, 0)

        for i in range(_PKW // _LANES):
            ones_v[pl.ds(i * _LANES, _LANES)] = one16

        cbase = sid * cnt_per_tile
        cfull, crem = divmod(cnt_per_tile, 2048)
        for k in range(cfull):
            pltpu.sync_copy(zf_v, cnt_sh.at[pl.ds(cbase + k * 2048, 2048)])
        if crem:
            pltpu.sync_copy(
                zf_v.at[pl.ds(0, crem)],
                cnt_sh.at[pl.ds(cbase + cfull * 2048, crem)],
            )

        pltpu.sync_copy(pk_hbm.at[pl.ds(wid * nchunks, nchunks)], pk_v)

        plsc.subcore_barrier()

        def _chunk(j, carry):
            for i in range(_PKW // _LANES):
                sl = pl.ds(i * _LANES, _LANES)
                p = pk_v[j, sl]
                tgt = lax.shift_right_logical(p, _NODE_BITS) & nmask
                et = lax.shift_right_logical(p, 2 * _NODE_BITS) & tmask
                cidx_v[sl] = tgt * n_types + et
            pltpu.sync_copy(ones_v, cnt_sh.at[cidx_v], add=True)
            return carry

        lax.fori_loop(0, nchunks, _chunk, 0)

        plsc.subcore_barrier()

        pltpu.sync_copy(
            cnt_sh.at[pl.ds(cbase, cnt_per_tile)],
            cnt_out.at[cid, pl.ds(cbase, cnt_per_tile)],
        )

    return sc_cnt


def kernel(node_features, edge_index, edge_type, W, b, edge_emb_table):
    n, cin = node_features.shape
    cout, _ = W.shape
    n_types = edge_emb_table.shape[0]
    e = edge_index.shape[1]
    assert n + 1 <= (1 << _NODE_BITS) and n_types <= (1 << (32 - 2 * _NODE_BITS))

    src = edge_index[0].astype(jnp.int32)
    tgt = edge_index[1].astype(jnp.int32)
    et = edge_type.astype(jnp.int32)

    npad = -(-(n + 1) // 128) * 128
    ept = -(-e // (_NUM_WORKERS * _PKW * 8)) * (_PKW * 8)
    epad = ept * _NUM_WORKERS
    pad = epad - e
    if pad:
        spread = jnp.arange(pad, dtype=jnp.int32)
        src = jnp.concatenate([src, spread % n])
        tgt = jnp.concatenate([tgt, n + spread % (npad - n)])
        et = jnp.concatenate([et, jnp.zeros((pad,), jnp.int32)])
    packed = src | (tgt << _NODE_BITS) | (et << (2 * _NODE_BITS))
    pk2 = packed.reshape(epad // _PKW, _PKW)

    transformed = _linear(node_features, W.T, b.reshape(1, cout), block_rows=2000)

    agg = _make_sc_agg(npad, ept // _CHUNK, cout)(transformed, pk2)
    cnt = _make_sc_cnt(npad, epad // _PKW // _NUM_WORKERS, n_types)(pk2)

    cnt3 = cnt.reshape(2, npad, n_types)
    return _combine(transformed, agg, cnt3, edge_emb_table, block_rows=2000)
